# Initial kernel scaffold; baseline (speedup 1.0000x reference)
import jax
import jax.numpy as jnp
from jax.experimental import pallas as pl


def kernel(x, token_type_ids, W_word, W_pos, W_tok, ln_gamma, ln_beta):
    raise NotImplementedError("write your pallas kernel here")


# speedup vs baseline: 16.5658x; 16.5658x over previous
import functools

import jax
import jax.numpy as jnp
from jax import lax
from jax.experimental import pallas as pl
from jax.experimental.pallas import tpu as pltpu
from jax.experimental.pallas import tpu_sc as plsc

VOCAB = 100000
HIDDEN = 128
EPS = 1e-06
B, L = 4096, 200
NTOK = B * L


_ROWS_BLK = 2000
_GRID = VOCAB // _ROWS_BLK
_XROWS = NTOK // HIDDEN
_XBLK = _XROWS // _GRID


def _table_body(w_ref, p_ref, tok_ref, g_ref, b_ref, x_ref, tt_ref,
                out_ref, cidx_ref):
    s = w_ref[...] + p_ref[...]
    gamma = g_ref[...]
    beta = b_ref[...]
    for t in range(2):
        h = s + tok_ref[t:t + 1, :]
        mu = jnp.mean(h, axis=1, keepdims=True)
        var = jnp.mean(jnp.square(h - mu), axis=1, keepdims=True)
        y = (h - mu) / jnp.sqrt(var + EPS) * gamma + beta
        out_ref[:, t * HIDDEN:(t + 1) * HIDDEN] = y
    cidx_ref[...] = x_ref[...] * 2 + tt_ref[...]


_table_call = pl.pallas_call(
    _table_body,
    grid=(_GRID,),
    in_specs=[
        pl.BlockSpec((_ROWS_BLK, HIDDEN), lambda g: (g, 0)),
        pl.BlockSpec((_ROWS_BLK, HIDDEN), lambda g: (g, 0)),
        pl.BlockSpec((2, HIDDEN), lambda g: (0, 0)),
        pl.BlockSpec((1, HIDDEN), lambda g: (0, 0)),
        pl.BlockSpec((1, HIDDEN), lambda g: (0, 0)),
        pl.BlockSpec((_XBLK, HIDDEN), lambda g: (g, 0)),
        pl.BlockSpec((_XBLK, HIDDEN), lambda g: (g, 0)),
    ],
    out_specs=[
        pl.BlockSpec((_ROWS_BLK, 2 * HIDDEN), lambda g: (g, 0)),
        pl.BlockSpec((_XBLK, HIDDEN), lambda g: (g, 0)),
    ],
    out_shape=[
        jax.ShapeDtypeStruct((VOCAB, 2 * HIDDEN), jnp.float32),
        jax.ShapeDtypeStruct((_XROWS, HIDDEN), jnp.int32),
    ],
)


_NC, _NS = 2, 16
_NW = _NC * _NS
_TPW = NTOK // _NW
_C = 512
_K = _C // 128
_NCHUNK = _TPW // _C

@functools.cache
def _build_gather_kernel():
    mesh = plsc.VectorSubcoreMesh(
        core_axis_name="c", subcore_axis_name="s",
        num_cores=_NC, num_subcores=_NS)

    @functools.partial(
        pl.kernel,
        out_type=jax.ShapeDtypeStruct((NTOK, HIDDEN), jnp.float32),
        mesh=mesh,
        scratch_types=[
            pltpu.VMEM((_K, 128), jnp.int32),
            pltpu.VMEM((_C, HIDDEN), jnp.float32),
            pltpu.SemaphoreType.DMA,
        ],
    )
    def gather_kernel(cidx_hbm, table_hbm, out_hbm, idx_v, rows_v, sem):
        wid = lax.axis_index("s") * _NC + lax.axis_index("c")

        def chunk(g, carry):
            base = wid * _TPW + g * _C
            pltpu.sync_copy(
                cidx_hbm.at[pl.ds(wid * (_TPW // 128) + g * _K, _K)], idx_v)
            descs = []
            for j in range(_K):
                descs.append(pltpu.async_copy(
                    table_hbm.at[idx_v.at[j]],
                    rows_v.at[pl.ds(j * 128, 128)], sem))
            for d in descs:
                d.wait()
            pltpu.sync_copy(rows_v, out_hbm.at[pl.ds(base, _C)])
            return carry

        lax.fori_loop(0, _NCHUNK, chunk, 0)

    return gather_kernel


def kernel(x, token_type_ids, W_word, W_pos, W_tok, ln_gamma, ln_beta):
    xr = x.reshape(_XROWS, HIDDEN)
    ttr = token_type_ids.reshape(_XROWS, HIDDEN)
    table, cidx = _table_call(
        W_word, W_pos, W_tok,
        ln_gamma.reshape(1, HIDDEN), ln_beta.reshape(1, HIDDEN), xr, ttr)
    out = _build_gather_kernel()(cidx, table.reshape(2 * VOCAB, HIDDEN))
    return out.reshape(B, L, HIDDEN)


# speedup vs baseline: 17.8008x; 1.0745x over previous
import functools

import jax
import jax.numpy as jnp
from jax import lax
from jax.experimental import pallas as pl
from jax.experimental.pallas import tpu as pltpu
from jax.experimental.pallas import tpu_sc as plsc

VOCAB = 100000
HIDDEN = 128
EPS = 1e-06
B, L = 4096, 200
NTOK = B * L


_ROWS_BLK = 2000
_GRID = VOCAB // _ROWS_BLK
_XROWS = NTOK // HIDDEN
_XBLK = _XROWS // _GRID


def _table_body(w_ref, p_ref, tok_ref, g_ref, b_ref, x_ref, tt_ref,
                out_ref, cidx_ref):
    s = w_ref[...] + p_ref[...]
    gamma = g_ref[...]
    beta = b_ref[...]
    for t in range(2):
        h = s + tok_ref[t:t + 1, :]
        mu = jnp.mean(h, axis=1, keepdims=True)
        var = jnp.mean(jnp.square(h - mu), axis=1, keepdims=True)
        y = (h - mu) / jnp.sqrt(var + EPS) * gamma + beta
        out_ref[:, t * HIDDEN:(t + 1) * HIDDEN] = y
    cidx_ref[...] = x_ref[...] * 2 + tt_ref[...]


_table_call = pl.pallas_call(
    _table_body,
    grid=(_GRID,),
    in_specs=[
        pl.BlockSpec((_ROWS_BLK, HIDDEN), lambda g: (g, 0)),
        pl.BlockSpec((_ROWS_BLK, HIDDEN), lambda g: (g, 0)),
        pl.BlockSpec((2, HIDDEN), lambda g: (0, 0)),
        pl.BlockSpec((1, HIDDEN), lambda g: (0, 0)),
        pl.BlockSpec((1, HIDDEN), lambda g: (0, 0)),
        pl.BlockSpec((_XBLK, HIDDEN), lambda g: (g, 0)),
        pl.BlockSpec((_XBLK, HIDDEN), lambda g: (g, 0)),
    ],
    out_specs=[
        pl.BlockSpec((_ROWS_BLK, 2 * HIDDEN), lambda g: (g, 0)),
        pl.BlockSpec((_XBLK, HIDDEN), lambda g: (g, 0)),
    ],
    out_shape=[
        jax.ShapeDtypeStruct((VOCAB, 2 * HIDDEN), jnp.float32),
        jax.ShapeDtypeStruct((_XROWS, HIDDEN), jnp.int32),
    ],
)


_NC, _NS = 2, 16
_NW = _NC * _NS
_TPW = NTOK // _NW
_C = 256
_K = _C // 128
_NCHUNK = _TPW // _C

@functools.cache
def _build_gather_kernel():
    mesh = plsc.VectorSubcoreMesh(
        core_axis_name="c", subcore_axis_name="s",
        num_cores=_NC, num_subcores=_NS)

    @functools.partial(
        pl.kernel,
        out_type=jax.ShapeDtypeStruct((NTOK, HIDDEN), jnp.float32),
        mesh=mesh,
        scratch_types=[
            pltpu.VMEM((_K, 128), jnp.int32),
            pltpu.VMEM((_K, 128), jnp.int32),
            pltpu.VMEM((_C, HIDDEN), jnp.float32),
            pltpu.VMEM((_C, HIDDEN), jnp.float32),
            pltpu.SemaphoreType.DMA,
            pltpu.SemaphoreType.DMA,
            pltpu.SemaphoreType.DMA,
            pltpu.SemaphoreType.DMA,
        ],
    )
    def gather_kernel(cidx_hbm, table_hbm, out_hbm,
                      idx0, idx1, rows0, rows1, gsem0, gsem1, osem0, osem1):
        wid = lax.axis_index("s") * _NC + lax.axis_index("c")
        idx = (idx0, idx1)
        rows = (rows0, rows1)
        gsem = (gsem0, gsem1)
        osem = (osem0, osem1)
        base0 = wid * _TPW
        ibase0 = wid * (_TPW // 128)

        def idx_load(g, b):
            pltpu.sync_copy(cidx_hbm.at[pl.ds(ibase0 + g * _K, _K)], idx[b])

        def fire_gather(b):
            for j in range(_K):
                pltpu.async_copy(table_hbm.at[idx[b].at[j]],
                                 rows[b].at[pl.ds(j * 128, 128)], gsem[b])

        def wait_gather(b):
            for j in range(_K):
                pltpu.make_async_copy(
                    table_hbm.at[idx[b].at[j]],
                    rows[b].at[pl.ds(j * 128, 128)], gsem[b]).wait()

        def fire_out(g, b):
            pltpu.async_copy(rows[b], out_hbm.at[pl.ds(base0 + g * _C, _C)],
                             osem[b])

        def wait_out(g, b):
            pltpu.make_async_copy(
                rows[b], out_hbm.at[pl.ds(base0 + g * _C, _C)],
                osem[b]).wait()

        idx_load(0, 0)
        fire_gather(0)
        idx_load(1, 1)
        fire_gather(1)
        wait_gather(0)
        fire_out(0, 0)
        idx_load(2, 0)

        def pair(p, carry):
            for (b, off) in ((1, 1), (0, 2)):
                g = 2 * p + off
                wait_out(g - 1, 1 - b)
                fire_gather(1 - b)
                wait_gather(b)
                fire_out(g, b)

                @pl.when(g + 2 < _NCHUNK)
                def _(b=b, g=g):
                    idx_load(g + 2, b)
            return carry

        lax.fori_loop(0, (_NCHUNK - 2) // 2, pair, 0)
        wait_out(_NCHUNK - 2, 0)
        wait_gather(1)
        fire_out(_NCHUNK - 1, 1)
        wait_out(_NCHUNK - 1, 1)

    return gather_kernel


def kernel(x, token_type_ids, W_word, W_pos, W_tok, ln_gamma, ln_beta):
    xr = x.reshape(_XROWS, HIDDEN)
    ttr = token_type_ids.reshape(_XROWS, HIDDEN)
    table, cidx = _table_call(
        W_word, W_pos, W_tok,
        ln_gamma.reshape(1, HIDDEN), ln_beta.reshape(1, HIDDEN), xr, ttr)
    out = _build_gather_kernel()(cidx, table.reshape(2 * VOCAB, HIDDEN))
    return out.reshape(B, L, HIDDEN)


# speedup vs baseline: 18.0062x; 1.0115x over previous
import functools

import jax
import jax.numpy as jnp
from jax import lax
from jax.experimental import pallas as pl
from jax.experimental.pallas import tpu as pltpu
from jax.experimental.pallas import tpu_sc as plsc

VOCAB = 100000
HIDDEN = 128
EPS = 1e-06
B, L = 4096, 200
NTOK = B * L


_ROWS_BLK = 2000
_GRID = VOCAB // _ROWS_BLK
_XROWS = NTOK // HIDDEN
_XBLK = _XROWS // _GRID


def _table_body(w_ref, p_ref, tok_ref, g_ref, b_ref, x_ref, tt_ref,
                out_ref, cidx_ref):
    s = w_ref[...] + p_ref[...]
    gamma = g_ref[...]
    beta = b_ref[...]
    inv_h = 1.0 / HIDDEN
    for t in range(2):
        h = s + tok_ref[t:t + 1, :]
        mu = jnp.sum(h, axis=1, keepdims=True) * inv_h
        ex2 = jnp.sum(h * h, axis=1, keepdims=True) * inv_h
        rinv = jax.lax.rsqrt(ex2 - mu * mu + EPS)
        y = (h - mu) * rinv * gamma + beta
        out_ref[:, t * HIDDEN:(t + 1) * HIDDEN] = y
    cidx_ref[...] = x_ref[...] * 2 + tt_ref[...]


_table_call = pl.pallas_call(
    _table_body,
    grid=(_GRID,),
    in_specs=[
        pl.BlockSpec((_ROWS_BLK, HIDDEN), lambda g: (g, 0)),
        pl.BlockSpec((_ROWS_BLK, HIDDEN), lambda g: (g, 0)),
        pl.BlockSpec((2, HIDDEN), lambda g: (0, 0)),
        pl.BlockSpec((1, HIDDEN), lambda g: (0, 0)),
        pl.BlockSpec((1, HIDDEN), lambda g: (0, 0)),
        pl.BlockSpec((_XBLK, HIDDEN), lambda g: (g, 0)),
        pl.BlockSpec((_XBLK, HIDDEN), lambda g: (g, 0)),
    ],
    out_specs=[
        pl.BlockSpec((_ROWS_BLK, 2 * HIDDEN), lambda g: (g, 0)),
        pl.BlockSpec((_XBLK, HIDDEN), lambda g: (g, 0)),
    ],
    out_shape=[
        jax.ShapeDtypeStruct((VOCAB, 2 * HIDDEN), jnp.float32),
        jax.ShapeDtypeStruct((_XROWS, HIDDEN), jnp.int32),
    ],
)


_NC, _NS = 2, 16
_NW = _NC * _NS
_TPW = NTOK // _NW
_C = 128
_NCHUNK = _TPW // _C
_NB = 4

@functools.cache
def _build_gather_kernel():
    mesh = plsc.VectorSubcoreMesh(
        core_axis_name="c", subcore_axis_name="s",
        num_cores=_NC, num_subcores=_NS)

    @functools.partial(
        pl.kernel,
        out_type=jax.ShapeDtypeStruct((NTOK, HIDDEN), jnp.float32),
        mesh=mesh,
        scratch_types=[
            pltpu.VMEM((_NCHUNK, 128), jnp.int32),
        ] + [pltpu.VMEM((_C, HIDDEN), jnp.float32) for _ in range(_NB)]
          + [pltpu.SemaphoreType.DMA for _ in range(2 * _NB)],
    )
    def gather_kernel(cidx_hbm, table_hbm, out_hbm, idxall, *bufs_and_sems):
        rows = bufs_and_sems[:_NB]
        gsem = bufs_and_sems[_NB:2 * _NB]
        osem = bufs_and_sems[2 * _NB:3 * _NB]
        wid = lax.axis_index("s") * _NC + lax.axis_index("c")
        base0 = wid * _TPW
        ibase0 = wid * _NCHUNK

        pltpu.sync_copy(cidx_hbm.at[pl.ds(ibase0, _NCHUNK)], idxall)

        def fire_gather(g, b):
            pltpu.async_copy(table_hbm.at[idxall.at[g]], rows[b], gsem[b])

        def wait_gather(g, b):
            pltpu.make_async_copy(table_hbm.at[idxall.at[g]], rows[b],
                                  gsem[b]).wait()

        def fire_out(g, b):
            pltpu.async_copy(rows[b], out_hbm.at[pl.ds(base0 + g * _C, _C)],
                             osem[b])

        def wait_out(g, b):
            pltpu.make_async_copy(
                rows[b], out_hbm.at[pl.ds(base0 + g * _C, _C)],
                osem[b]).wait()

        for g0 in range(_NB - 1):
            fire_gather(g0, g0)

        def group(p, carry):
            for r in range(_NB):
                g = p * _NB + r
                bf = (r - 1) % _NB

                @pl.when(g >= 1)
                def _(g=g, bf=bf):
                    wait_out(g - 1, bf)

                @pl.when(g + _NB - 1 < _NCHUNK)
                def _(g=g, bf=bf):
                    fire_gather(g + _NB - 1, bf)

                wait_gather(g, r)
                fire_out(g, r)
            return carry

        lax.fori_loop(0, _NCHUNK // _NB, group, 0)
        wait_out(_NCHUNK - 1, (_NCHUNK - 1) % _NB)

    return gather_kernel


def kernel(x, token_type_ids, W_word, W_pos, W_tok, ln_gamma, ln_beta):
    xr = x.reshape(_XROWS, HIDDEN)
    ttr = token_type_ids.reshape(_XROWS, HIDDEN)
    table, cidx = _table_call(
        W_word, W_pos, W_tok,
        ln_gamma.reshape(1, HIDDEN), ln_beta.reshape(1, HIDDEN), xr, ttr)
    out = _build_gather_kernel()(cidx, table.reshape(2 * VOCAB, HIDDEN))
    return out.reshape(B, L, HIDDEN)


# speedup vs baseline: 19.4628x; 1.0809x over previous
import functools

import jax
import jax.numpy as jnp
from jax import lax
from jax.experimental import pallas as pl
from jax.experimental.pallas import tpu as pltpu
from jax.experimental.pallas import tpu_sc as plsc

VOCAB = 100000
HIDDEN = 128
EPS = 1e-06
B, L = 4096, 200
NTOK = B * L


_ROWS_BLK = 2000
_NBLK = VOCAB // _ROWS_BLK
_GRID = 2 * _NBLK
_XROWS = NTOK // HIDDEN
_XBLK = _XROWS // _GRID


def _table_body(w_ref, p_ref, tok_ref, g_ref, b_ref, x_ref, tt_ref,
                out_ref, cidx_ref):
    t = pl.program_id(0) % 2
    s = w_ref[...] + p_ref[...]
    gamma = g_ref[...]
    beta = b_ref[...]
    inv_h = 1.0 / HIDDEN
    tok = jnp.where(t == 0, tok_ref[0:1, :], tok_ref[1:2, :])
    h = s + tok
    mu = jnp.sum(h, axis=1, keepdims=True) * inv_h
    ex2 = jnp.sum(h * h, axis=1, keepdims=True) * inv_h
    rinv = jax.lax.rsqrt(ex2 - mu * mu + EPS)
    out_ref[...] = (h - mu) * rinv * gamma + beta
    cidx_ref[...] = tt_ref[...] * VOCAB + x_ref[...]


_table_call = pl.pallas_call(
    _table_body,
    grid=(_GRID,),
    in_specs=[
        pl.BlockSpec((_ROWS_BLK, HIDDEN), lambda g: (g // 2, 0)),
        pl.BlockSpec((_ROWS_BLK, HIDDEN), lambda g: (g // 2, 0)),
        pl.BlockSpec((2, HIDDEN), lambda g: (0, 0)),
        pl.BlockSpec((1, HIDDEN), lambda g: (0, 0)),
        pl.BlockSpec((1, HIDDEN), lambda g: (0, 0)),
        pl.BlockSpec((_XBLK, HIDDEN), lambda g: (g, 0)),
        pl.BlockSpec((_XBLK, HIDDEN), lambda g: (g, 0)),
    ],
    out_specs=[
        pl.BlockSpec((_ROWS_BLK, HIDDEN),
                     lambda g: ((g % 2) * _NBLK + g // 2, 0)),
        pl.BlockSpec((_XBLK, HIDDEN), lambda g: (g, 0)),
    ],
    out_shape=[
        jax.ShapeDtypeStruct((2 * VOCAB, HIDDEN), jnp.float32),
        jax.ShapeDtypeStruct((_XROWS, HIDDEN), jnp.int32),
    ],
)


_NC, _NS = 2, 16
_NW = _NC * _NS
_TPW = NTOK // _NW
_C = 128
_NCHUNK = _TPW // _C
_NB = 4

@functools.cache
def _build_gather_kernel():
    mesh = plsc.VectorSubcoreMesh(
        core_axis_name="c", subcore_axis_name="s",
        num_cores=_NC, num_subcores=_NS)

    @functools.partial(
        pl.kernel,
        out_type=jax.ShapeDtypeStruct((NTOK, HIDDEN), jnp.float32),
        mesh=mesh,
        scratch_types=[
            pltpu.VMEM((_NCHUNK, 128), jnp.int32),
        ] + [pltpu.VMEM((_C, HIDDEN), jnp.float32) for _ in range(_NB)]
          + [pltpu.SemaphoreType.DMA for _ in range(2 * _NB)],
    )
    def gather_kernel(cidx_hbm, table_hbm, out_hbm, idxall, *bufs_and_sems):
        rows = bufs_and_sems[:_NB]
        gsem = bufs_and_sems[_NB:2 * _NB]
        osem = bufs_and_sems[2 * _NB:3 * _NB]
        wid = lax.axis_index("s") * _NC + lax.axis_index("c")
        base0 = wid * _TPW
        ibase0 = wid * _NCHUNK

        pltpu.sync_copy(cidx_hbm.at[pl.ds(ibase0, _NCHUNK)], idxall)

        def fire_gather(g, b):
            pltpu.async_copy(table_hbm.at[idxall.at[g]], rows[b], gsem[b])

        def wait_gather(g, b):
            pltpu.make_async_copy(table_hbm.at[idxall.at[g]], rows[b],
                                  gsem[b]).wait()

        def fire_out(g, b):
            pltpu.async_copy(rows[b], out_hbm.at[pl.ds(base0 + g * _C, _C)],
                             osem[b])

        def wait_out(g, b):
            pltpu.make_async_copy(
                rows[b], out_hbm.at[pl.ds(base0 + g * _C, _C)],
                osem[b]).wait()

        for g0 in range(_NB - 1):
            fire_gather(g0, g0)

        def group(p, carry):
            for r in range(_NB):
                g = p * _NB + r
                bf = (r - 1) % _NB

                @pl.when(g >= 1)
                def _(g=g, bf=bf):
                    wait_out(g - 1, bf)

                @pl.when(g + _NB - 1 < _NCHUNK)
                def _(g=g, bf=bf):
                    fire_gather(g + _NB - 1, bf)

                wait_gather(g, r)
                fire_out(g, r)
            return carry

        lax.fori_loop(0, _NCHUNK // _NB, group, 0)
        wait_out(_NCHUNK - 1, (_NCHUNK - 1) % _NB)

    return gather_kernel


def kernel(x, token_type_ids, W_word, W_pos, W_tok, ln_gamma, ln_beta):
    xr = x.reshape(_XROWS, HIDDEN)
    ttr = token_type_ids.reshape(_XROWS, HIDDEN)
    table, cidx = _table_call(
        W_word, W_pos, W_tok,
        ln_gamma.reshape(1, HIDDEN), ln_beta.reshape(1, HIDDEN), xr, ttr)
    out = _build_gather_kernel()(cidx, table)
    return out.reshape(B, L, HIDDEN)


# speedup vs baseline: 21.5077x; 1.1051x over previous
import functools

import jax
import jax.numpy as jnp
from jax import lax
from jax.experimental import pallas as pl
from jax.experimental.pallas import tpu as pltpu
from jax.experimental.pallas import tpu_sc as plsc

VOCAB = 100000
HIDDEN = 128
EPS = 1e-06
B, L = 4096, 200
NTOK = B * L


_ROWS_BLK = 10000
_NBLK = VOCAB // _ROWS_BLK
_GRID = 2 * _NBLK
_XROWS = NTOK // HIDDEN
_XBLK = _XROWS // _GRID


def _table_body(w_ref, p_ref, tok_ref, g_ref, b_ref, x_ref, tt_ref,
                out_ref, cidx_ref):
    t = pl.program_id(0) % 2
    s = w_ref[...] + p_ref[...]
    gamma = g_ref[...]
    beta = b_ref[...]
    inv_h = 1.0 / HIDDEN
    tok = jnp.where(t == 0, tok_ref[0:1, :], tok_ref[1:2, :])
    h = s + tok
    mu = jnp.sum(h, axis=1, keepdims=True) * inv_h
    ex2 = jnp.sum(h * h, axis=1, keepdims=True) * inv_h
    rinv = jax.lax.rsqrt(ex2 - mu * mu + EPS)
    out_ref[...] = (h - mu) * rinv * gamma + beta
    cidx_ref[...] = tt_ref[...] * VOCAB + x_ref[...]


_table_call = pl.pallas_call(
    _table_body,
    grid=(_GRID,),
    in_specs=[
        pl.BlockSpec((_ROWS_BLK, HIDDEN), lambda g: (g // 2, 0)),
        pl.BlockSpec((_ROWS_BLK, HIDDEN), lambda g: (g // 2, 0)),
        pl.BlockSpec((2, HIDDEN), lambda g: (0, 0)),
        pl.BlockSpec((1, HIDDEN), lambda g: (0, 0)),
        pl.BlockSpec((1, HIDDEN), lambda g: (0, 0)),
        pl.BlockSpec((_XBLK, HIDDEN), lambda g: (g, 0)),
        pl.BlockSpec((_XBLK, HIDDEN), lambda g: (g, 0)),
    ],
    out_specs=[
        pl.BlockSpec((_ROWS_BLK, HIDDEN),
                     lambda g: ((g % 2) * _NBLK + g // 2, 0)),
        pl.BlockSpec((_XBLK, HIDDEN), lambda g: (g, 0)),
    ],
    out_shape=[
        jax.ShapeDtypeStruct((2 * VOCAB, HIDDEN), jnp.float32),
        jax.ShapeDtypeStruct((_XROWS, HIDDEN), jnp.int32),
    ],
)


_NC, _NS = 2, 16
_NW = _NC * _NS
_TPW = NTOK // _NW
_C = 128
_NCHUNK = _TPW // _C
_NB = 4

@functools.cache
def _build_gather_kernel():
    mesh = plsc.VectorSubcoreMesh(
        core_axis_name="c", subcore_axis_name="s",
        num_cores=_NC, num_subcores=_NS)

    @functools.partial(
        pl.kernel,
        out_type=jax.ShapeDtypeStruct((NTOK, HIDDEN), jnp.float32),
        mesh=mesh,
        scratch_types=[
            pltpu.VMEM((_NCHUNK, 128), jnp.int32),
        ] + [pltpu.VMEM((_C, HIDDEN), jnp.float32) for _ in range(_NB)]
          + [pltpu.SemaphoreType.DMA for _ in range(2 * _NB)],
    )
    def gather_kernel(cidx_hbm, table_hbm, out_hbm, idxall, *bufs_and_sems):
        rows = bufs_and_sems[:_NB]
        gsem = bufs_and_sems[_NB:2 * _NB]
        osem = bufs_and_sems[2 * _NB:3 * _NB]
        wid = lax.axis_index("s") * _NC + lax.axis_index("c")
        base0 = wid * _TPW
        ibase0 = wid * _NCHUNK

        pltpu.sync_copy(cidx_hbm.at[pl.ds(ibase0, _NCHUNK)], idxall)

        def fire_gather(g, b):
            pltpu.async_copy(table_hbm.at[idxall.at[g]], rows[b], gsem[b])

        def wait_gather(g, b):
            pltpu.make_async_copy(table_hbm.at[idxall.at[g]], rows[b],
                                  gsem[b]).wait()

        def fire_out(g, b):
            pltpu.async_copy(rows[b], out_hbm.at[pl.ds(base0 + g * _C, _C)],
                             osem[b])

        def wait_out(g, b):
            pltpu.make_async_copy(
                rows[b], out_hbm.at[pl.ds(base0 + g * _C, _C)],
                osem[b]).wait()

        for g0 in range(_NB - 1):
            fire_gather(g0, g0)

        def group(p, carry):
            for r in range(_NB):
                g = p * _NB + r
                bf = (r - 1) % _NB

                @pl.when(g >= 1)
                def _(g=g, bf=bf):
                    wait_out(g - 1, bf)

                @pl.when(g + _NB - 1 < _NCHUNK)
                def _(g=g, bf=bf):
                    fire_gather(g + _NB - 1, bf)

                wait_gather(g, r)
                fire_out(g, r)
            return carry

        lax.fori_loop(0, _NCHUNK // _NB, group, 0)
        wait_out(_NCHUNK - 1, (_NCHUNK - 1) % _NB)

    return gather_kernel


def kernel(x, token_type_ids, W_word, W_pos, W_tok, ln_gamma, ln_beta):
    xr = x.reshape(_XROWS, HIDDEN)
    ttr = token_type_ids.reshape(_XROWS, HIDDEN)
    table, cidx = _table_call(
        W_word, W_pos, W_tok,
        ln_gamma.reshape(1, HIDDEN), ln_beta.reshape(1, HIDDEN), xr, ttr)
    out = _build_gather_kernel()(cidx, table)
    return out.reshape(B, L, HIDDEN)


# speedup vs baseline: 23.0179x; 1.0702x over previous
import functools

import jax
import jax.numpy as jnp
from jax import lax
from jax.experimental import pallas as pl
from jax.experimental.pallas import tpu as pltpu
from jax.experimental.pallas import tpu_sc as plsc

VOCAB = 100000
HIDDEN = 128
EPS = 1e-06
B, L = 4096, 200
NTOK = B * L


_ROWS_BLK = 5000
_NBLK = VOCAB // _ROWS_BLK
_XROWS = NTOK // HIDDEN
_XBLK = _XROWS // _NBLK


def _table_body(w_ref, p_ref, tok_ref, g_ref, b_ref, x_ref, tt_ref,
                tbl_ref, cidx_ref, scr, sem):
    g = pl.program_id(0)
    par = jax.lax.rem(g, 2)
    R = _ROWS_BLK

    @pl.when(g >= 2)
    def _():
        for t in range(2):
            pltpu.make_async_copy(
                scr.at[par, t],
                tbl_ref.at[pl.ds(t * VOCAB + (g - 2) * R, R)],
                sem.at[par]).wait()

    s = w_ref[...] + p_ref[...]
    gamma = g_ref[...]
    beta = b_ref[...]
    inv_h = 1.0 / HIDDEN
    for t in range(2):
        h = s + tok_ref[t:t + 1, :]
        mu = jnp.sum(h, axis=1, keepdims=True) * inv_h
        ex2 = jnp.sum(h * h, axis=1, keepdims=True) * inv_h
        rinv = jax.lax.rsqrt(ex2 - mu * mu + EPS)
        scr[par, t] = (h - mu) * rinv * gamma + beta
        pltpu.async_copy(scr.at[par, t],
                         tbl_ref.at[pl.ds(t * VOCAB + g * R, R)],
                         sem.at[par])
    cidx_ref[...] = tt_ref[...] * VOCAB + x_ref[...]

    @pl.when(g == _NBLK - 1)
    def _():
        for gg_off in (1, 0):
            p2 = jax.lax.rem(g - gg_off, 2)
            for t in range(2):
                pltpu.make_async_copy(
                    scr.at[p2, t],
                    tbl_ref.at[pl.ds(t * VOCAB + (g - gg_off) * R, R)],
                    sem.at[p2]).wait()


_table_call = pl.pallas_call(
    _table_body,
    grid=(_NBLK,),
    in_specs=[
        pl.BlockSpec((_ROWS_BLK, HIDDEN), lambda g: (g, 0)),
        pl.BlockSpec((_ROWS_BLK, HIDDEN), lambda g: (g, 0)),
        pl.BlockSpec((2, HIDDEN), lambda g: (0, 0)),
        pl.BlockSpec((1, HIDDEN), lambda g: (0, 0)),
        pl.BlockSpec((1, HIDDEN), lambda g: (0, 0)),
        pl.BlockSpec((_XBLK, HIDDEN), lambda g: (g, 0)),
        pl.BlockSpec((_XBLK, HIDDEN), lambda g: (g, 0)),
    ],
    out_specs=[
        pl.BlockSpec(memory_space=pl.ANY),
        pl.BlockSpec((_XBLK, HIDDEN), lambda g: (g, 0)),
    ],
    out_shape=[
        jax.ShapeDtypeStruct((2 * VOCAB, HIDDEN), jnp.float32),
        jax.ShapeDtypeStruct((_XROWS, HIDDEN), jnp.int32),
    ],
    scratch_shapes=[
        pltpu.VMEM((2, 2, _ROWS_BLK, HIDDEN), jnp.float32),
        pltpu.SemaphoreType.DMA((2,)),
    ],
)


_NC, _NS = 2, 16
_NW = _NC * _NS
_TPW = NTOK // _NW
_C = 128
_NCHUNK = _TPW // _C
_NB = 4

@functools.cache
def _build_gather_kernel():
    mesh = plsc.VectorSubcoreMesh(
        core_axis_name="c", subcore_axis_name="s",
        num_cores=_NC, num_subcores=_NS)

    @functools.partial(
        pl.kernel,
        out_type=jax.ShapeDtypeStruct((NTOK, HIDDEN), jnp.float32),
        mesh=mesh,
        scratch_types=[
            pltpu.VMEM((_NCHUNK, 128), jnp.int32),
        ] + [pltpu.VMEM((_C, HIDDEN), jnp.float32) for _ in range(_NB)]
          + [pltpu.SemaphoreType.DMA for _ in range(2 * _NB)],
    )
    def gather_kernel(cidx_hbm, table_hbm, out_hbm, idxall, *bufs_and_sems):
        rows = bufs_and_sems[:_NB]
        gsem = bufs_and_sems[_NB:2 * _NB]
        osem = bufs_and_sems[2 * _NB:3 * _NB]
        wid = lax.axis_index("s") * _NC + lax.axis_index("c")
        base0 = wid * _TPW
        ibase0 = wid * _NCHUNK

        pltpu.sync_copy(cidx_hbm.at[pl.ds(ibase0, _NCHUNK)], idxall)

        def fire_gather(g, b):
            pltpu.async_copy(table_hbm.at[idxall.at[g]], rows[b], gsem[b])

        def wait_gather(g, b):
            pltpu.make_async_copy(table_hbm.at[idxall.at[g]], rows[b],
                                  gsem[b]).wait()

        def fire_out(g, b):
            pltpu.async_copy(rows[b], out_hbm.at[pl.ds(base0 + g * _C, _C)],
                             osem[b])

        def wait_out(g, b):
            pltpu.make_async_copy(
                rows[b], out_hbm.at[pl.ds(base0 + g * _C, _C)],
                osem[b]).wait()

        for g0 in range(_NB - 1):
            fire_gather(g0, g0)

        def group(p, carry):
            for r in range(_NB):
                g = p * _NB + r
                bf = (r - 1) % _NB

                @pl.when(g >= 1)
                def _(g=g, bf=bf):
                    wait_out(g - 1, bf)

                @pl.when(g + _NB - 1 < _NCHUNK)
                def _(g=g, bf=bf):
                    fire_gather(g + _NB - 1, bf)

                wait_gather(g, r)
                fire_out(g, r)
            return carry

        lax.fori_loop(0, _NCHUNK // _NB, group, 0)
        wait_out(_NCHUNK - 1, (_NCHUNK - 1) % _NB)

    return gather_kernel


def kernel(x, token_type_ids, W_word, W_pos, W_tok, ln_gamma, ln_beta):
    xr = x.reshape(_XROWS, HIDDEN)
    ttr = token_type_ids.reshape(_XROWS, HIDDEN)
    table, cidx = _table_call(
        W_word, W_pos, W_tok,
        ln_gamma.reshape(1, HIDDEN), ln_beta.reshape(1, HIDDEN), xr, ttr)
    out = _build_gather_kernel()(cidx, table)
    return out.reshape(B, L, HIDDEN)


# speedup vs baseline: 23.0778x; 1.0026x over previous
import functools

import jax
import jax.numpy as jnp
from jax import lax
from jax.experimental import pallas as pl
from jax.experimental.pallas import tpu as pltpu
from jax.experimental.pallas import tpu_sc as plsc

VOCAB = 100000
HIDDEN = 128
EPS = 1e-06
B, L = 4096, 200
NTOK = B * L


_ROWS_BLK = 5000
_NBLK = VOCAB // _ROWS_BLK
_XROWS = NTOK // HIDDEN
_XBLK = _XROWS // _NBLK


def _table_body(w_ref, p_ref, tok_ref, g_ref, b_ref, x_ref, tt_ref,
                tbl_ref, cidx_ref, scr, sem):
    g = pl.program_id(0)
    par = jax.lax.rem(g, 2)
    R = _ROWS_BLK

    @pl.when(g >= 2)
    def _():
        for t in range(2):
            pltpu.make_async_copy(
                scr.at[par, t],
                tbl_ref.at[pl.ds(t * VOCAB + (g - 2) * R, R)],
                sem.at[par]).wait()

    s = w_ref[...] + p_ref[...]
    gamma = g_ref[...]
    beta = b_ref[...]
    inv_h = 1.0 / HIDDEN
    for t in range(2):
        h = s + tok_ref[t:t + 1, :]
        mu = jnp.sum(h, axis=1, keepdims=True) * inv_h
        ex2 = jnp.sum(h * h, axis=1, keepdims=True) * inv_h
        rinv = jax.lax.rsqrt(ex2 - mu * mu + EPS)
        scr[par, t] = (h - mu) * rinv * gamma + beta
        pltpu.async_copy(scr.at[par, t],
                         tbl_ref.at[pl.ds(t * VOCAB + g * R, R)],
                         sem.at[par])
    cidx_ref[...] = tt_ref[...] * VOCAB + x_ref[...]

    @pl.when(g == _NBLK - 1)
    def _():
        for gg_off in (1, 0):
            p2 = jax.lax.rem(g - gg_off, 2)
            for t in range(2):
                pltpu.make_async_copy(
                    scr.at[p2, t],
                    tbl_ref.at[pl.ds(t * VOCAB + (g - gg_off) * R, R)],
                    sem.at[p2]).wait()


_table_call = pl.pallas_call(
    _table_body,
    grid=(_NBLK,),
    in_specs=[
        pl.BlockSpec((_ROWS_BLK, HIDDEN), lambda g: (g, 0)),
        pl.BlockSpec((_ROWS_BLK, HIDDEN), lambda g: (g, 0)),
        pl.BlockSpec((2, HIDDEN), lambda g: (0, 0)),
        pl.BlockSpec((1, HIDDEN), lambda g: (0, 0)),
        pl.BlockSpec((1, HIDDEN), lambda g: (0, 0)),
        pl.BlockSpec((_XBLK, HIDDEN), lambda g: (g, 0)),
        pl.BlockSpec((_XBLK, HIDDEN), lambda g: (g, 0)),
    ],
    out_specs=[
        pl.BlockSpec(memory_space=pl.ANY),
        pl.BlockSpec((_XBLK, HIDDEN), lambda g: (g, 0)),
    ],
    out_shape=[
        jax.ShapeDtypeStruct((2 * VOCAB, HIDDEN), jnp.float32),
        jax.ShapeDtypeStruct((_XROWS, HIDDEN), jnp.int32),
    ],
    scratch_shapes=[
        pltpu.VMEM((2, 2, _ROWS_BLK, HIDDEN), jnp.float32),
        pltpu.SemaphoreType.DMA((2,)),
    ],
)


_NC, _NS = 2, 16
_NW = _NC * _NS
_TPW = NTOK // _NW
_C = 128
_K = _C // 128
_NCHUNK = _TPW // _C
_NIDX = _TPW // 128
_NB = 4

@functools.cache
def _build_gather_kernel():
    mesh = plsc.VectorSubcoreMesh(
        core_axis_name="c", subcore_axis_name="s",
        num_cores=_NC, num_subcores=_NS)

    @functools.partial(
        pl.kernel,
        out_type=jax.ShapeDtypeStruct((NTOK, HIDDEN), jnp.float32),
        mesh=mesh,
        scratch_types=[
            pltpu.VMEM((_NIDX, 128), jnp.int32),
        ] + [pltpu.VMEM((_C, HIDDEN), jnp.float32) for _ in range(_NB)]
          + [pltpu.SemaphoreType.DMA for _ in range(2 * _NB)],
    )
    def gather_kernel(cidx_hbm, table_hbm, out_hbm, idxall, *bufs_and_sems):
        rows = bufs_and_sems[:_NB]
        gsem = bufs_and_sems[_NB:2 * _NB]
        osem = bufs_and_sems[2 * _NB:3 * _NB]
        wid = lax.axis_index("s") * _NC + lax.axis_index("c")
        base0 = wid * _TPW
        ibase0 = wid * _NIDX

        pltpu.sync_copy(cidx_hbm.at[pl.ds(ibase0, _NIDX)], idxall)

        def fire_gather(g, b):
            pltpu.async_copy(table_hbm.at[idxall.at[g]], rows[b], gsem[b])

        def wait_gather(g, b):
            pltpu.make_async_copy(table_hbm.at[idxall.at[g]], rows[b],
                                  gsem[b]).wait()

        def fire_out(g, b):
            pltpu.async_copy(rows[b], out_hbm.at[pl.ds(base0 + g * _C, _C)],
                             osem[b])

        def wait_out(g, b):
            pltpu.make_async_copy(
                rows[b], out_hbm.at[pl.ds(base0 + g * _C, _C)],
                osem[b]).wait()

        def step(g, b, bf):
            @pl.when(g >= 1)
            def _():
                wait_out(g - 1, bf)

            @pl.when(g + _NB - 1 < _NCHUNK)
            def _():
                fire_gather(g + _NB - 1, bf)

            wait_gather(g, b)
            fire_out(g, b)

        for g0 in range(_NB - 1):
            fire_gather(g0, g0)

        def group(p, carry):
            for r in range(_NB):
                step(p * _NB + r, r, (r - 1) % _NB)
            return carry

        _NFULL = _NCHUNK // _NB
        lax.fori_loop(0, _NFULL, group, 0)
        for g in range(_NFULL * _NB, _NCHUNK):
            step(g, g % _NB, (g - 1) % _NB)
        wait_out(_NCHUNK - 1, (_NCHUNK - 1) % _NB)

    return gather_kernel


def kernel(x, token_type_ids, W_word, W_pos, W_tok, ln_gamma, ln_beta):
    xr = x.reshape(_XROWS, HIDDEN)
    ttr = token_type_ids.reshape(_XROWS, HIDDEN)
    table, cidx = _table_call(
        W_word, W_pos, W_tok,
        ln_gamma.reshape(1, HIDDEN), ln_beta.reshape(1, HIDDEN), xr, ttr)
    out = _build_gather_kernel()(cidx, table)
    return out.reshape(B, L, HIDDEN)
